# initial kernel scaffold (unmeasured)
import jax
import jax.numpy as jnp
from jax import lax
from jax.experimental import pallas as pl
from jax.experimental.pallas import tpu as pltpu

N_DEV = 4
E_TOTAL = 16
E_LOCAL = 4


def kernel(x, router_W, route_idx, expert_W):
    m, d = x.shape
    _, _, h_dim = expert_W.shape

    def body(x_ref, rw_ref, idx_ref, ew_ref, out_ref,
             x_all, w_all, rs_send, rs_recv,
             agx_send, agx_recv, agw_send, agw_recv,
             rs_send_sem, rs_recv_sems):
        p = lax.axis_index("i")
        right = lax.rem(p + 1, N_DEV)
        left = lax.rem(p + N_DEV - 1, N_DEV)

        barrier = pltpu.get_barrier_semaphore()
        for nbr in (left, right):
            pl.semaphore_signal(barrier, inc=1, device_id=(nbr,),
                                device_id_type=pl.DeviceIdType.MESH)
        pl.semaphore_wait(barrier, 2)

        scores = jnp.dot(x_ref[...], rw_ref[...],
                         preferred_element_type=jnp.float32)
        smax = jnp.max(scores, axis=-1, keepdims=True)
        pexp = jnp.exp(scores - smax)
        e0 = idx_ref[:, 0:1]
        e1 = idx_ref[:, 1:2]
        iota = lax.broadcasted_iota(jnp.int32, (m, E_TOTAL), 1)
        oh0 = iota == e0
        oh1 = iota == e1
        g0 = jnp.sum(jnp.where(oh0, pexp, 0.0), axis=-1, keepdims=True)
        g1 = jnp.sum(jnp.where(oh1, pexp, 0.0), axis=-1, keepdims=True)
        w = (jnp.where(oh0, g0, 0.0) + jnp.where(oh1, g1, 0.0)) / (g0 + g1)

        x_all[0] = x_ref[...]
        w_all[0] = w

        for h in range(N_DEV - 1):
            rx = pltpu.make_async_remote_copy(
                src_ref=x_all.at[h], dst_ref=x_all.at[h + 1],
                send_sem=agx_send.at[h], recv_sem=agx_recv.at[h],
                device_id=(right,), device_id_type=pl.DeviceIdType.MESH)
            rw_c = pltpu.make_async_remote_copy(
                src_ref=w_all.at[h], dst_ref=w_all.at[h + 1],
                send_sem=agw_send.at[h], recv_sem=agw_recv.at[h],
                device_id=(right,), device_id_type=pl.DeviceIdType.MESH)
            rx.start()
            rw_c.start()
            rx.wait()
            rw_c.wait()

        def partial(s):
            xb = x_all[s]
            wloc = pl.load(
                w_all, (s, slice(None), pl.ds(p * E_LOCAL, E_LOCAL)))
            acc = None
            for el in range(E_LOCAL):
                xw = xb * wloc[:, el:el + 1]
                t = jnp.dot(xw, ew_ref[el],
                            preferred_element_type=jnp.float32)
                acc = t if acc is None else acc + t
            return acc

        for h in range(N_DEV - 1):
            val = partial(h + 1)
            if h > 0:
                val = val + rs_recv[h - 1]
            rs_send[0] = val
            r = pltpu.make_async_remote_copy(
                src_ref=rs_send.at[0], dst_ref=rs_recv.at[h],
                send_sem=rs_send_sem, recv_sem=rs_recv_sems.at[h],
                device_id=(right,), device_id_type=pl.DeviceIdType.MESH)
            r.start()
            r.wait()

        out_ref[...] = partial(0) + rs_recv[N_DEV - 2]

    return pl.pallas_call(
        body,
        out_shape=jax.ShapeDtypeStruct((m, h_dim), jnp.float32),
        in_specs=[
            pl.BlockSpec(memory_space=pltpu.VMEM),
            pl.BlockSpec(memory_space=pltpu.VMEM),
            pl.BlockSpec(memory_space=pltpu.VMEM),
            pl.BlockSpec(memory_space=pltpu.VMEM),
        ],
        out_specs=pl.BlockSpec(memory_space=pltpu.VMEM),
        scratch_shapes=[
            pltpu.VMEM((N_DEV, m, d), jnp.float32),
            pltpu.VMEM((N_DEV, m, E_TOTAL), jnp.float32),
            pltpu.VMEM((1, m, h_dim), jnp.float32),
            pltpu.VMEM((N_DEV - 1, m, h_dim), jnp.float32),
            pltpu.SemaphoreType.DMA((N_DEV - 1,)),
            pltpu.SemaphoreType.DMA((N_DEV - 1,)),
            pltpu.SemaphoreType.DMA((N_DEV - 1,)),
            pltpu.SemaphoreType.DMA((N_DEV - 1,)),
            pltpu.SemaphoreType.DMA,
            pltpu.SemaphoreType.DMA((N_DEV - 1,)),
        ],
        compiler_params=pltpu.CompilerParams(collective_id=0),
    )(x, router_W, route_idx, expert_W)


# baseline (device time: 262855 ns/iter reference)
import jax
import jax.numpy as jnp
from jax import lax
from jax.experimental import pallas as pl
from jax.experimental.pallas import tpu as pltpu

N_DEV = 4
E_TOTAL = 16
E_LOCAL = 4


def kernel(x, router_W, route_idx, expert_W):
    m, d = x.shape
    _, _, h_dim = expert_W.shape

    def body(x_ref, rw_ref, idx_ref, ew_ref, out_ref,
             x_all, w_all, rs_send, rs_recv,
             agx_send, agx_recv, agw_send, agw_recv,
             rs_send_sem, rs_recv_sems):
        p = lax.axis_index("i")
        right = lax.rem(p + 1, N_DEV)
        left = lax.rem(p + N_DEV - 1, N_DEV)

        barrier = pltpu.get_barrier_semaphore()
        for nbr in (left, right):
            pl.semaphore_signal(barrier, inc=1, device_id=(nbr,),
                                device_id_type=pl.DeviceIdType.MESH)
        pl.semaphore_wait(barrier, 2)

        scores = jnp.dot(x_ref[...], rw_ref[...],
                         preferred_element_type=jnp.float32)
        smax = jnp.max(scores, axis=-1, keepdims=True)
        pexp = jnp.exp(scores - smax)
        e0 = idx_ref[:, 0:1]
        e1 = idx_ref[:, 1:2]
        iota = lax.broadcasted_iota(jnp.int32, (m, E_TOTAL), 1)
        oh0 = iota == e0
        oh1 = iota == e1
        g0 = jnp.sum(jnp.where(oh0, pexp, 0.0), axis=-1, keepdims=True)
        g1 = jnp.sum(jnp.where(oh1, pexp, 0.0), axis=-1, keepdims=True)
        w = (jnp.where(oh0, g0, 0.0) + jnp.where(oh1, g1, 0.0)) / (g0 + g1)

        x_all[0] = x_ref[...]
        w_all[0] = w

        for h in range(N_DEV - 1):
            rx = pltpu.make_async_remote_copy(
                src_ref=x_all.at[h], dst_ref=x_all.at[h + 1],
                send_sem=agx_send.at[h], recv_sem=agx_recv.at[h],
                device_id=(right,), device_id_type=pl.DeviceIdType.MESH)
            rw_c = pltpu.make_async_remote_copy(
                src_ref=w_all.at[h], dst_ref=w_all.at[h + 1],
                send_sem=agw_send.at[h], recv_sem=agw_recv.at[h],
                device_id=(right,), device_id_type=pl.DeviceIdType.MESH)
            rx.start()
            rw_c.start()
            rx.wait()
            rw_c.wait()

        def partial(s):
            xb = x_all[s]
            wb = w_all[s]
            acc = None
            for el in range(E_LOCAL):
                ge = p * E_LOCAL + el
                wcol = jnp.sum(jnp.where(iota == ge, wb, 0.0),
                               axis=-1, keepdims=True)
                xw = xb * wcol
                t = jnp.dot(xw, ew_ref[el],
                            preferred_element_type=jnp.float32)
                acc = t if acc is None else acc + t
            return acc

        for h in range(N_DEV - 1):
            val = partial(h + 1)
            if h > 0:
                val = val + rs_recv[h - 1]
            rs_send[0] = val
            r = pltpu.make_async_remote_copy(
                src_ref=rs_send.at[0], dst_ref=rs_recv.at[h],
                send_sem=rs_send_sem, recv_sem=rs_recv_sems.at[h],
                device_id=(right,), device_id_type=pl.DeviceIdType.MESH)
            r.start()
            r.wait()

        out_ref[...] = partial(0) + rs_recv[N_DEV - 2]

    return pl.pallas_call(
        body,
        out_shape=jax.ShapeDtypeStruct((m, h_dim), jnp.float32),
        in_specs=[
            pl.BlockSpec(memory_space=pltpu.VMEM),
            pl.BlockSpec(memory_space=pltpu.VMEM),
            pl.BlockSpec(memory_space=pltpu.VMEM),
            pl.BlockSpec(memory_space=pltpu.VMEM),
        ],
        out_specs=pl.BlockSpec(memory_space=pltpu.VMEM),
        scratch_shapes=[
            pltpu.VMEM((N_DEV, m, d), jnp.float32),
            pltpu.VMEM((N_DEV, m, E_TOTAL), jnp.float32),
            pltpu.VMEM((1, m, h_dim), jnp.float32),
            pltpu.VMEM((N_DEV - 1, m, h_dim), jnp.float32),
            pltpu.SemaphoreType.DMA((N_DEV - 1,)),
            pltpu.SemaphoreType.DMA((N_DEV - 1,)),
            pltpu.SemaphoreType.DMA((N_DEV - 1,)),
            pltpu.SemaphoreType.DMA((N_DEV - 1,)),
            pltpu.SemaphoreType.DMA,
            pltpu.SemaphoreType.DMA((N_DEV - 1,)),
        ],
        compiler_params=pltpu.CompilerParams(collective_id=0),
    )(x, router_W, route_idx, expert_W)


# device time: 138633 ns/iter; 1.8960x vs baseline; 1.8960x over previous
import jax
import jax.numpy as jnp
from jax import lax
from jax.experimental import pallas as pl
from jax.experimental.pallas import tpu as pltpu

N_DEV = 4
E_TOTAL = 16
E_LOCAL = 4


def kernel(x, router_W, route_idx, expert_W):
    m, d = x.shape
    _, _, h_dim = expert_W.shape

    def body(x_ref, rw_ref, idx_ref, ew_ref, out_ref,
             x_all, w_all, ew_bf, rs_send, rs_recv,
             agx_send, agx_recv, agw_send, agw_recv,
             rs_send_sems, rs_recv_sems):
        p = lax.axis_index("i")
        right = lax.rem(p + 1, N_DEV)
        left = lax.rem(p + N_DEV - 1, N_DEV)

        barrier = pltpu.get_barrier_semaphore()
        for nbr in (left, right):
            pl.semaphore_signal(barrier, inc=1, device_id=(nbr,),
                                device_id_type=pl.DeviceIdType.MESH)
        pl.semaphore_wait(barrier, 2)

        scores = jnp.dot(x_ref[...], rw_ref[...],
                         preferred_element_type=jnp.float32)
        smax = jnp.max(scores, axis=-1, keepdims=True)
        pexp = jnp.exp(scores - smax)
        e0 = idx_ref[:, 0:1]
        e1 = idx_ref[:, 1:2]
        iota = lax.broadcasted_iota(jnp.int32, (m, E_TOTAL), 1)
        oh0 = iota == e0
        oh1 = iota == e1
        g0 = jnp.sum(jnp.where(oh0, pexp, 0.0), axis=-1, keepdims=True)
        g1 = jnp.sum(jnp.where(oh1, pexp, 0.0), axis=-1, keepdims=True)
        w = (jnp.where(oh0, g0, 0.0) + jnp.where(oh1, g1, 0.0)) / (g0 + g1)

        x_all[0] = x_ref[...].astype(jnp.bfloat16)
        w_all[0] = w
        ew_bf[...] = ew_ref[...].astype(jnp.bfloat16)

        def ag_hop(h):
            rx = pltpu.make_async_remote_copy(
                src_ref=x_all.at[h], dst_ref=x_all.at[h + 1],
                send_sem=agx_send.at[h], recv_sem=agx_recv.at[h],
                device_id=(right,), device_id_type=pl.DeviceIdType.MESH)
            rw_c = pltpu.make_async_remote_copy(
                src_ref=w_all.at[h], dst_ref=w_all.at[h + 1],
                send_sem=agw_send.at[h], recv_sem=agw_recv.at[h],
                device_id=(right,), device_id_type=pl.DeviceIdType.MESH)
            rx.start()
            rw_c.start()
            return rx, rw_c

        def rs_hop(h):
            r = pltpu.make_async_remote_copy(
                src_ref=rs_send.at[h], dst_ref=rs_recv.at[h],
                send_sem=rs_send_sems.at[h], recv_sem=rs_recv_sems.at[h],
                device_id=(right,), device_id_type=pl.DeviceIdType.MESH)
            r.start()
            return r

        def partial(s):
            xb = x_all[s]
            wb = w_all[s]
            acc = None
            for el in range(E_LOCAL):
                ge = p * E_LOCAL + el
                wcol = jnp.sum(jnp.where(iota == ge, wb, 0.0),
                               axis=-1, keepdims=True)
                xw = xb * wcol.astype(jnp.bfloat16)
                t = jnp.dot(xw, ew_bf[el],
                            preferred_element_type=jnp.float32)
                acc = t if acc is None else acc + t
            return acc

        ag0 = ag_hop(0)
        own = partial(0)
        for a in ag0:
            a.wait()
        ag1 = ag_hop(1)

        val = partial(1)
        rs_send[0] = val.astype(jnp.bfloat16)
        rs0 = rs_hop(0)
        for a in ag1:
            a.wait()
        ag2 = ag_hop(2)

        val = partial(2)
        rs0.wait()
        rs_send[1] = (val + rs_recv[0].astype(jnp.float32)
                      ).astype(jnp.bfloat16)
        rs1 = rs_hop(1)
        for a in ag2:
            a.wait()

        val = partial(3)
        rs1.wait()
        rs_send[2] = (val + rs_recv[1].astype(jnp.float32)
                      ).astype(jnp.bfloat16)
        rs2 = rs_hop(2)

        rs2.wait()
        out_ref[...] = own + rs_recv[2].astype(jnp.float32)

    return pl.pallas_call(
        body,
        out_shape=jax.ShapeDtypeStruct((m, h_dim), jnp.float32),
        in_specs=[
            pl.BlockSpec(memory_space=pltpu.VMEM),
            pl.BlockSpec(memory_space=pltpu.VMEM),
            pl.BlockSpec(memory_space=pltpu.VMEM),
            pl.BlockSpec(memory_space=pltpu.VMEM),
        ],
        out_specs=pl.BlockSpec(memory_space=pltpu.VMEM),
        scratch_shapes=[
            pltpu.VMEM((N_DEV, m, d), jnp.bfloat16),
            pltpu.VMEM((N_DEV, m, E_TOTAL), jnp.float32),
            pltpu.VMEM((E_LOCAL, d, h_dim), jnp.bfloat16),
            pltpu.VMEM((N_DEV - 1, m, h_dim), jnp.bfloat16),
            pltpu.VMEM((N_DEV - 1, m, h_dim), jnp.bfloat16),
            pltpu.SemaphoreType.DMA((N_DEV - 1,)),
            pltpu.SemaphoreType.DMA((N_DEV - 1,)),
            pltpu.SemaphoreType.DMA((N_DEV - 1,)),
            pltpu.SemaphoreType.DMA((N_DEV - 1,)),
            pltpu.SemaphoreType.DMA((N_DEV - 1,)),
            pltpu.SemaphoreType.DMA((N_DEV - 1,)),
        ],
        compiler_params=pltpu.CompilerParams(collective_id=0),
    )(x, router_W, route_idx, expert_W)


# device time: 29196 ns/iter; 9.0031x vs baseline; 4.7484x over previous
import jax
import jax.numpy as jnp
from jax import lax
from jax.experimental import pallas as pl
from jax.experimental.pallas import tpu as pltpu

N_DEV = 4
E_TOTAL = 16
E_LOCAL = 4


def kernel(x, router_W, route_idx, expert_W):
    m, d = x.shape
    _, _, h_dim = expert_W.shape
    mh = m // 2

    def body(x_ref, rw_ref, idx_ref, ew_ref, out_ref,
             xR, wR, xL, wL, ew_bf,
             rs_sendR, rs_recvR, rs_sendL, rs_recvL,
             agx_sendR, agx_recvR, agw_sendR, agw_recvR,
             agx_sendL, agx_recvL, agw_sendL, agw_recvL,
             rs_send_semsR, rs_recv_semsR, rs_send_semsL, rs_recv_semsL):
        p = lax.axis_index("i")
        right = lax.rem(p + 1, N_DEV)
        left = lax.rem(p + N_DEV - 1, N_DEV)

        barrier = pltpu.get_barrier_semaphore()
        for nbr in (left, right):
            pl.semaphore_signal(barrier, inc=1, device_id=(nbr,),
                                device_id_type=pl.DeviceIdType.MESH)
        pl.semaphore_wait(barrier, 2)

        scores = jnp.dot(x_ref[...], rw_ref[...],
                         preferred_element_type=jnp.float32)
        smax = jnp.max(scores, axis=-1, keepdims=True)
        pexp = jnp.exp(scores - smax)
        e0 = idx_ref[:, 0:1]
        e1 = idx_ref[:, 1:2]
        iota_m = lax.broadcasted_iota(jnp.int32, (m, E_TOTAL), 1)
        oh0 = iota_m == e0
        oh1 = iota_m == e1
        g0 = jnp.sum(jnp.where(oh0, pexp, 0.0), axis=-1, keepdims=True)
        g1 = jnp.sum(jnp.where(oh1, pexp, 0.0), axis=-1, keepdims=True)
        w = (jnp.where(oh0, g0, 0.0) + jnp.where(oh1, g1, 0.0)) / (g0 + g1)

        xb16 = x_ref[...].astype(jnp.bfloat16)
        xR[0] = xb16[:mh]
        xL[0] = xb16[mh:]
        wR[0] = w[:mh]
        wL[0] = w[mh:]
        ew_bf[...] = ew_ref[...].astype(jnp.bfloat16)

        def ag_hop(h, x_buf, w_buf, xs, xr, ws, wr, nbr):
            rx = pltpu.make_async_remote_copy(
                src_ref=x_buf.at[h], dst_ref=x_buf.at[h + 1],
                send_sem=xs.at[h], recv_sem=xr.at[h],
                device_id=(nbr,), device_id_type=pl.DeviceIdType.MESH)
            rw_c = pltpu.make_async_remote_copy(
                src_ref=w_buf.at[h], dst_ref=w_buf.at[h + 1],
                send_sem=ws.at[h], recv_sem=wr.at[h],
                device_id=(nbr,), device_id_type=pl.DeviceIdType.MESH)
            rx.start()
            rw_c.start()
            return rx, rw_c

        def rs_hop(h, snd, rcv, ss, rs_, nbr):
            r = pltpu.make_async_remote_copy(
                src_ref=snd.at[h], dst_ref=rcv.at[h],
                send_sem=ss.at[h], recv_sem=rs_.at[h],
                device_id=(nbr,), device_id_type=pl.DeviceIdType.MESH)
            r.start()
            return r

        iota_h = lax.broadcasted_iota(jnp.int32, (mh, E_TOTAL), 1)

        def partial(x_buf, w_buf, s):
            xb = x_buf[s]
            wb = w_buf[s]
            acc = None
            for el in range(E_LOCAL):
                ge = p * E_LOCAL + el
                wcol = jnp.sum(jnp.where(iota_h == ge, wb, 0.0),
                               axis=-1, keepdims=True)
                xw = xb * wcol.astype(jnp.bfloat16)
                t = jnp.dot(xw, ew_bf[el],
                            preferred_element_type=jnp.float32)
                acc = t if acc is None else acc + t
            return acc

        def agR(h):
            return ag_hop(h, xR, wR, agx_sendR, agx_recvR,
                          agw_sendR, agw_recvR, right)

        def agL(h):
            return ag_hop(h, xL, wL, agx_sendL, agx_recvL,
                          agw_sendL, agw_recvL, left)

        agr = agR(0)
        agl = agL(0)
        ownR = partial(xR, wR, 0)
        ownL = partial(xL, wL, 0)
        for a in agr:
            a.wait()
        agr = agR(1)

        vR = partial(xR, wR, 1)
        rs_sendR[0] = vR.astype(jnp.bfloat16)
        rsR = rs_hop(0, rs_sendR, rs_recvR, rs_send_semsR,
                     rs_recv_semsR, right)
        for a in agl:
            a.wait()
        agl = agL(1)

        vL = partial(xL, wL, 1)
        rs_sendL[0] = vL.astype(jnp.bfloat16)
        rsL = rs_hop(0, rs_sendL, rs_recvL, rs_send_semsL,
                     rs_recv_semsL, left)
        for a in agr:
            a.wait()
        agr = agR(2)

        vR = partial(xR, wR, 2)
        rsR.wait()
        rs_sendR[1] = (vR + rs_recvR[0].astype(jnp.float32)
                       ).astype(jnp.bfloat16)
        rsR = rs_hop(1, rs_sendR, rs_recvR, rs_send_semsR,
                     rs_recv_semsR, right)
        for a in agl:
            a.wait()
        agl = agL(2)

        vL = partial(xL, wL, 2)
        rsL.wait()
        rs_sendL[1] = (vL + rs_recvL[0].astype(jnp.float32)
                       ).astype(jnp.bfloat16)
        rsL = rs_hop(1, rs_sendL, rs_recvL, rs_send_semsL,
                     rs_recv_semsL, left)
        for a in agr:
            a.wait()

        vR = partial(xR, wR, 3)
        rsR.wait()
        rs_sendR[2] = (vR + rs_recvR[1].astype(jnp.float32)
                       ).astype(jnp.bfloat16)
        rsR = rs_hop(2, rs_sendR, rs_recvR, rs_send_semsR,
                     rs_recv_semsR, right)
        for a in agl:
            a.wait()

        vL = partial(xL, wL, 3)
        rsL.wait()
        rs_sendL[2] = (vL + rs_recvL[1].astype(jnp.float32)
                       ).astype(jnp.bfloat16)
        rsL = rs_hop(2, rs_sendL, rs_recvL, rs_send_semsL,
                     rs_recv_semsL, left)

        rsR.wait()
        out_ref[:mh] = ownR + rs_recvR[2].astype(jnp.float32)
        rsL.wait()
        out_ref[mh:] = ownL + rs_recvL[2].astype(jnp.float32)

    dma3 = pltpu.SemaphoreType.DMA((N_DEV - 1,))
    return pl.pallas_call(
        body,
        out_shape=jax.ShapeDtypeStruct((m, h_dim), jnp.float32),
        in_specs=[
            pl.BlockSpec(memory_space=pltpu.VMEM),
            pl.BlockSpec(memory_space=pltpu.VMEM),
            pl.BlockSpec(memory_space=pltpu.VMEM),
            pl.BlockSpec(memory_space=pltpu.VMEM),
        ],
        out_specs=pl.BlockSpec(memory_space=pltpu.VMEM),
        scratch_shapes=[
            pltpu.VMEM((N_DEV, mh, d), jnp.bfloat16),
            pltpu.VMEM((N_DEV, mh, E_TOTAL), jnp.float32),
            pltpu.VMEM((N_DEV, mh, d), jnp.bfloat16),
            pltpu.VMEM((N_DEV, mh, E_TOTAL), jnp.float32),
            pltpu.VMEM((E_LOCAL, d, h_dim), jnp.bfloat16),
            pltpu.VMEM((N_DEV - 1, mh, h_dim), jnp.bfloat16),
            pltpu.VMEM((N_DEV - 1, mh, h_dim), jnp.bfloat16),
            pltpu.VMEM((N_DEV - 1, mh, h_dim), jnp.bfloat16),
            pltpu.VMEM((N_DEV - 1, mh, h_dim), jnp.bfloat16),
            dma3, dma3, dma3, dma3,
            dma3, dma3, dma3, dma3,
            dma3, dma3,
            dma3, dma3,
        ],
        compiler_params=pltpu.CompilerParams(collective_id=0),
    )(x, router_W, route_idx, expert_W)
